# Initial kernel scaffold; baseline (speedup 1.0000x reference)
import jax
import jax.numpy as jnp
from jax.experimental import pallas as pl


def kernel(params, x, edge_index, edge_attr, batch):
    raise NotImplementedError("write your pallas kernel here")


# speedup vs baseline: 5.4061x; 5.4061x over previous
import functools

import jax
import jax.numpy as jnp
from jax import lax
from jax.experimental import pallas as pl
from jax.experimental.pallas import tpu as pltpu
from jax.experimental.pallas import tpu_sc as plsc

N_NODES = 10000
N_EDGES = 160000
EMB = 128
HID = 32
HEADS = 4
ROW = 144
K = 64
N_CHUNKS = N_EDGES // K
N_WORKERS = 32
T_PER_W = (N_CHUNKS + N_WORKERS - 1) // N_WORKERS
N_PAD = 10240
ROWS_PER_TILE = N_PAD // 16

_f32 = jnp.float32
_i32 = jnp.int32


def _sc_edge_body(src_h, dst_h, a_h, xp_h, ep_h, z_h, out_h,
                  srcv, dstv, arow_d, arow_s, xj, eprow, msg,
                  aggr_sh, sem1, sem2, sem3):
    cid = lax.axis_index("c")
    sid = lax.axis_index("s")
    wid = sid * 2 + cid

    pltpu.sync_copy(z_h, aggr_sh.at[pl.ds(sid * ROWS_PER_TILE, ROWS_PER_TILE)])
    plsc.subcore_barrier()

    zero16 = jnp.zeros((16,), _f32)
    for kk in range(K // 16):
        idx = lax.iota(_i32, 16) + (kk * 16)
        for c in range(132, ROW):
            plsc.store_scatter(msg, [idx, jnp.full((16,), c, _i32)], zero16)

    def chunk(t, carry):
        g = t * N_WORKERS + wid

        @pl.when(g < N_CHUNKS)
        def _():
            base = g * K
            pltpu.sync_copy(src_h.at[pl.ds(base, K)], srcv)
            pltpu.sync_copy(dst_h.at[pl.ds(base, K)], dstv)
            cp1 = pltpu.async_copy(xp_h.at[srcv], xj, sem1)
            cp2 = pltpu.async_copy(a_h.at[dstv], arow_d, sem2)
            cp3 = pltpu.async_copy(a_h.at[srcv], arow_s, sem3)
            pltpu.sync_copy(ep_h.at[pl.ds(base, K)], eprow)
            cp1.wait()
            cp2.wait()
            cp3.wait()
            for kk in range(K // 16):
                idx = lax.iota(_i32, 16) + (kk * 16)
                for h in range(HEADS):
                    ad = plsc.load_gather(arow_d, [idx, jnp.full((16,), h, _i32)])
                    asr = plsc.load_gather(arow_s, [idx, jnp.full((16,), 4 + h, _i32)])
                    ae = plsc.load_gather(eprow, [idx, jnp.full((16,), 128 + h, _i32)])
                    al = ad + asr + ae
                    al = jnp.where(al >= 0.0, al, al * 0.2)
                    ex = jnp.exp(al)
                    plsc.store_scatter(msg, [idx, jnp.full((16,), 128 + h, _i32)], ex)

            def per_edge(k, c2):
                ev = msg[k, pl.ds(128, 16)]
                for h in range(HEADS):
                    exs = ev[h]
                    for c in range(2):
                        sl = pl.ds(h * 32 + c * 16, 16)
                        msg[k, sl] = xj[k, sl] * eprow[k, sl] * exs
                return c2

            lax.fori_loop(0, K, per_edge, 0)

            pltpu.sync_copy(msg, aggr_sh.at[dstv], add=True)

        return carry

    lax.fori_loop(0, T_PER_W, chunk, 0)
    plsc.subcore_barrier()

    pltpu.sync_copy(aggr_sh.at[pl.ds(sid * ROWS_PER_TILE, ROWS_PER_TILE)],
                    out_h.at[cid, pl.ds(sid * ROWS_PER_TILE, ROWS_PER_TILE)])


@functools.cache
def _make_sc_edge():
    return functools.partial(
        pl.kernel,
        out_type=jax.ShapeDtypeStruct((2, N_PAD, ROW), _f32),
        mesh=plsc.VectorSubcoreMesh(core_axis_name="c", subcore_axis_name="s"),
        scratch_types=[
            pltpu.VMEM((K,), _i32),
            pltpu.VMEM((K,), _i32),
            pltpu.VMEM((K, 16), _f32),
            pltpu.VMEM((K, 16), _f32),
            pltpu.VMEM((K, 128), _f32),
            pltpu.VMEM((K, ROW), _f32),
            pltpu.VMEM((K, ROW), _f32),
            pltpu.VMEM_SHARED((N_PAD, ROW), _f32),
            pltpu.SemaphoreType.DMA,
            pltpu.SemaphoreType.DMA,
            pltpu.SemaphoreType.DMA,
        ],
        compiler_params=pltpu.CompilerParams(use_tc_tiling_on_sc=False,
                                             needs_layout_passes=False),
    )(_sc_edge_body)


def _sc_edge(*args):
    return _make_sc_edge()(*args)


_RT = 1000


def _celu(x):
    return jnp.where(x > 0.0, x, jnp.exp(x) - 1.0)


def _init_nodes_body(xi_ref, xemb_ref, l0w_ref, l0b_ref, wn_ref, wf_ref,
                     x0_ref, xp_ref, a_ref):
    xi = xi_ref[...]
    ii = lax.broadcasted_iota(_i32, (_RT, 178), 1)
    oh = jnp.zeros((_RT, 178), _f32)
    for j in range(9):
        oh = oh + (xi[:, j:j + 1] == ii).astype(_f32)
    xe = jnp.dot(oh, xemb_ref[...], preferred_element_type=_f32)
    x0 = _celu(jnp.dot(xe, l0w_ref[...], preferred_element_type=_f32)
               + l0b_ref[...])
    x0_ref[...] = x0
    xp_ref[...] = jnp.dot(x0, wn_ref[...], preferred_element_type=_f32)
    a_ref[...] = jnp.dot(x0, wf_ref[...], preferred_element_type=_f32)


def _tc_init_nodes(x_idx, x_emb, l0w, l0b, wn1, wf1):
    n_t = N_NODES // _RT
    w_spec = lambda shp: pl.BlockSpec(shp, lambda i: (0, 0))
    return pl.pallas_call(
        _init_nodes_body,
        grid=(n_t,),
        in_specs=[
            pl.BlockSpec((_RT, 9), lambda i: (i, 0)),
            w_spec((178, EMB)),
            w_spec((EMB, HID)),
            w_spec((1, HID)),
            w_spec((HID, EMB)),
            w_spec((HID, 16)),
        ],
        out_specs=[
            pl.BlockSpec((_RT, HID), lambda i: (i, 0)),
            pl.BlockSpec((_RT, EMB), lambda i: (i, 0)),
            pl.BlockSpec((_RT, 16), lambda i: (i, 0)),
        ],
        out_shape=[
            jax.ShapeDtypeStruct((N_NODES, HID), _f32),
            jax.ShapeDtypeStruct((N_NODES, EMB), _f32),
            jax.ShapeDtypeStruct((N_NODES, 16), _f32),
        ],
    )(x_idx, x_emb, l0w, l0b, wn1, wf1)


_RE = 2000


def _init_edges_body(ea_ref, t1_ref, ae1_ref, t2_ref, ae2_ref,
                     ep1_ref, ep2_ref):
    ea = ea_ref[...]
    ii = lax.broadcasted_iota(_i32, (_RE, 18), 1)
    oh = jnp.zeros((_RE, 18), _f32)
    for j in range(3):
        oh = oh + (ea[:, j:j + 1] == ii).astype(_f32)
    for t_ref, ae_ref, out_ref in ((t1_ref, ae1_ref, ep1_ref),
                                   (t2_ref, ae2_ref, ep2_ref)):
        ep = jnp.dot(oh, t_ref[...], preferred_element_type=_f32)
        ae = jnp.dot(oh, ae_ref[...], preferred_element_type=_f32)
        out_ref[...] = jnp.concatenate([ep, ae], axis=1)


def _tc_init_edges(edge_attr, t1, ae1, t2, ae2):
    n_t = N_EDGES // _RE
    w_spec = lambda shp: pl.BlockSpec(shp, lambda i: (0, 0))
    return pl.pallas_call(
        _init_edges_body,
        grid=(n_t,),
        in_specs=[
            pl.BlockSpec((_RE, 3), lambda i: (i, 0)),
            w_spec((18, EMB)),
            w_spec((18, 16)),
            w_spec((18, EMB)),
            w_spec((18, 16)),
        ],
        out_specs=[
            pl.BlockSpec((_RE, ROW), lambda i: (i, 0)),
            pl.BlockSpec((_RE, ROW), lambda i: (i, 0)),
        ],
        out_shape=[
            jax.ShapeDtypeStruct((N_EDGES, ROW), _f32),
            jax.ShapeDtypeStruct((N_EDGES, ROW), _f32),
        ],
    )(edge_attr, t1, ae1, t2, ae2)


def _step_body(use_xnext, a0_ref, a1_ref, hn_ref, xin_ref,
               ws_ref, bias_ref,
               wir_ref, wiz_ref, win_ref, whr_ref, whz_ref, whn_ref,
               br_ref, bz_ref, bin_ref, bhn_ref,
               lng_ref, lnb_ref, wnn_ref, wfn_ref,
               hn2_ref, xn_ref, xp_ref, a_ref):
    a0 = a0_ref[...]
    a1 = a1_ref[...]
    s = a0[:, 128:ROW] + a1[:, 128:ROW]
    winv = 1.0 / (s + 1e-16)
    r16 = lax.broadcasted_iota(_i32, (16, EMB), 0)
    c16 = lax.broadcasted_iota(_i32, (16, EMB), 1) // 32
    erep = (r16 == c16).astype(_f32)
    wexp = jnp.dot(winv, erep, preferred_element_type=_f32)
    aggr = (a0[:, :128] + a1[:, :128]) * wexp
    mm = _celu(jnp.dot(aggr, ws_ref[...], preferred_element_type=_f32)
               + bias_ref[...])
    hn = hn_ref[...]
    rr = jax.nn.sigmoid(jnp.dot(mm, wir_ref[...], preferred_element_type=_f32)
                        + jnp.dot(hn, whr_ref[...], preferred_element_type=_f32)
                        + br_ref[...])
    zz = jax.nn.sigmoid(jnp.dot(mm, wiz_ref[...], preferred_element_type=_f32)
                        + jnp.dot(hn, whz_ref[...], preferred_element_type=_f32)
                        + bz_ref[...])
    nn = jnp.tanh(jnp.dot(mm, win_ref[...], preferred_element_type=_f32)
                  + bin_ref[...]
                  + rr * (jnp.dot(hn, whn_ref[...], preferred_element_type=_f32)
                          + bhn_ref[...]))
    hn2 = (1.0 - zz) * nn + zz * hn
    mu = jnp.mean(hn2, axis=-1, keepdims=True)
    var = jnp.mean((hn2 - mu) ** 2, axis=-1, keepdims=True)
    cur = (hn2 - mu) / jnp.sqrt(var + 1e-5) * lng_ref[...] + lnb_ref[...]
    xn = xin_ref[...] + cur
    hn2_ref[...] = hn2
    xn_ref[...] = xn
    xsrc = xn if use_xnext else cur
    xp_ref[...] = jnp.dot(xsrc, wnn_ref[...], preferred_element_type=_f32)
    a_ref[...] = jnp.dot(xsrc, wfn_ref[...], preferred_element_type=_f32)


def _tc_step(use_xnext, a0, a1, hn, xin, ws, bias, gru6, b4, lng, lnb,
             wnn, wfn):
    n_t = N_NODES // _RT
    w_spec = lambda shp: pl.BlockSpec(shp, lambda i: (0, 0))
    r_spec = lambda d: pl.BlockSpec((_RT, d), lambda i: (i, 0))
    return pl.pallas_call(
        functools.partial(_step_body, use_xnext),
        grid=(n_t,),
        in_specs=[
            r_spec(ROW), r_spec(ROW), r_spec(HID), r_spec(HID),
            w_spec((EMB, HID)), w_spec((1, HID)),
            w_spec((HID, HID)), w_spec((HID, HID)), w_spec((HID, HID)),
            w_spec((HID, HID)), w_spec((HID, HID)), w_spec((HID, HID)),
            w_spec((1, HID)), w_spec((1, HID)), w_spec((1, HID)),
            w_spec((1, HID)),
            w_spec((1, HID)), w_spec((1, HID)),
            w_spec((HID, EMB)), w_spec((HID, 16)),
        ],
        out_specs=[r_spec(HID), r_spec(HID), r_spec(EMB), r_spec(16)],
        out_shape=[
            jax.ShapeDtypeStruct((N_NODES, HID), _f32),
            jax.ShapeDtypeStruct((N_NODES, HID), _f32),
            jax.ShapeDtypeStruct((N_NODES, EMB), _f32),
            jax.ShapeDtypeStruct((N_NODES, 16), _f32),
        ],
    )(a0, a1, hn, xin, ws, bias, *gru6, *b4, lng, lnb, wnn, wfn)


def _s2s_body(xf_ref, b2_ref,
              wii_ref, wif_ref, wig_ref, wio_ref,
              whi_ref, whf_ref, whg_ref, who_ref,
              bi_ref, bf_ref, bg_ref, bo_ref,
              o1w_ref, o1b_ref, olng_ref, olnb_ref, o2w_ref, o2b_ref,
              out_ref):
    xf = xf_ref[...]
    b2 = b2_ref[...]
    gi = lax.broadcasted_iota(_i32, (N_NODES, 64), 1)
    mask = (b2 == gi).astype(_f32)
    q_star = jnp.zeros((64, 64), _f32)
    h = jnp.zeros((64, HID), _f32)
    c = jnp.zeros((64, HID), _f32)
    for _ in range(3):
        g_i = jax.nn.sigmoid(jnp.dot(q_star, wii_ref[...], preferred_element_type=_f32)
                             + jnp.dot(h, whi_ref[...], preferred_element_type=_f32)
                             + bi_ref[...])
        g_f = jax.nn.sigmoid(jnp.dot(q_star, wif_ref[...], preferred_element_type=_f32)
                             + jnp.dot(h, whf_ref[...], preferred_element_type=_f32)
                             + bf_ref[...])
        g_g = jnp.tanh(jnp.dot(q_star, wig_ref[...], preferred_element_type=_f32)
                       + jnp.dot(h, whg_ref[...], preferred_element_type=_f32)
                       + bg_ref[...])
        g_o = jax.nn.sigmoid(jnp.dot(q_star, wio_ref[...], preferred_element_type=_f32)
                             + jnp.dot(h, who_ref[...], preferred_element_type=_f32)
                             + bo_ref[...])
        c = g_f * c + g_i * g_g
        h = g_o * jnp.tanh(c)
        emat = lax.dot_general(xf, h, (((1,), (1,)), ((), ())),
                               preferred_element_type=_f32)
        e = jnp.sum(mask * emat, axis=1, keepdims=True)
        m = jnp.max(jnp.where(mask > 0.0, emat, -1e30), axis=0,
                    keepdims=True)
        mg = jnp.sum(mask * m, axis=1, keepdims=True)
        ex = jnp.exp(e - mg)
        ssum = jnp.sum(mask * ex, axis=0, keepdims=True)
        sg = jnp.sum(mask * ssum, axis=1, keepdims=True)
        a = ex / (sg + 1e-16)
        r = lax.dot_general(mask, a * xf, (((0,), (0,)), ((), ())),
                            preferred_element_type=_f32)
        q_star = jnp.concatenate([h, r], axis=1)
    hf = jnp.dot(q_star, o1w_ref[...], preferred_element_type=_f32) + o1b_ref[...]
    mu = jnp.mean(hf, axis=-1, keepdims=True)
    var = jnp.mean((hf - mu) ** 2, axis=-1, keepdims=True)
    hf = (hf - mu) / jnp.sqrt(var + 1e-5) * olng_ref[...] + olnb_ref[...]
    hf = jnp.maximum(hf, 0.0)
    out_ref[...] = jnp.dot(hf, o2w_ref[...], preferred_element_type=_f32) + o2b_ref[...]


def _tc_s2s(xf, batch2, s2s_w, o1w, o1b, olng, olnb, o2w, o2b):
    return pl.pallas_call(
        _s2s_body,
        out_shape=jax.ShapeDtypeStruct((64, 2), _f32),
    )(xf, batch2, *s2s_w, o1w, o1b, olng, olnb, o2w, o2b)


def kernel(params, x, edge_index, edge_attr, batch):
    src = edge_index[0]
    dst = edge_index[1]

    blocks = params["blocks"]
    folded = []
    for blk in blocks:
        wn = blk["wn"]
        watt = blk["watt"]
        w_i, w_e, w_j = watt[:, :HID], watt[:, HID:2 * HID], watt[:, 2 * HID:]
        wn3 = wn.reshape(HID, HEADS, HID)
        wf_i = jnp.einsum("khd,hd->kh", wn3, w_i)
        wf_j = jnp.einsum("khd,hd->kh", wn3, w_j)
        wf = jnp.concatenate([wf_i, wf_j, jnp.zeros((HID, 8), _f32)], axis=1)
        t_b = params["e_emb"] @ blk["we"]
        ae_b = jnp.einsum("vhd,hd->vh", t_b.reshape(18, HEADS, HID), w_e)
        ae_b = jnp.concatenate([ae_b, jnp.zeros((18, 12), _f32)], axis=1)
        bih = blk["gru_bih"]
        bhh = blk["gru_bhh"]
        gru6 = tuple(w[:, i * HID:(i + 1) * HID]
                     for w in (blk["gru_wih"], blk["gru_whh"]) for i in range(3))
        gru6 = (gru6[0], gru6[1], gru6[2], gru6[3], gru6[4], gru6[5])
        b4 = (
            (bih[:HID] + bhh[:HID]).reshape(1, HID),
            (bih[HID:2 * HID] + bhh[HID:2 * HID]).reshape(1, HID),
            bih[2 * HID:].reshape(1, HID),
            bhh[2 * HID:].reshape(1, HID),
        )
        folded.append(dict(
            wn=wn, wf=wf, t=t_b, ae=ae_b,
            ws=blk["ws"], bias=blk["bias"].reshape(1, HID),
            gru6=gru6, b4=b4,
            lng=blk["ln_g"].reshape(1, HID), lnb=blk["ln_b"].reshape(1, HID),
        ))

    s2s_w = []
    for w, d in ((params["s2s_wih"], 2 * HID), (params["s2s_whh"], HID)):
        for i in range(4):
            s2s_w.append(w[:, i * HID:(i + 1) * HID])
    for i in range(4):
        s2s_w.append(params["s2s_b"][i * HID:(i + 1) * HID].reshape(1, HID))

    zeros_tile = jnp.zeros((ROWS_PER_TILE, ROW), _f32)
    batch2 = batch.reshape(N_NODES, 1)

    x0, xp, a_tab = _tc_init_nodes(
        x, params["x_emb"], params["lin0_w"],
        params["lin0_b"].reshape(1, HID), folded[0]["wn"], folded[0]["wf"])
    ep1, ep2 = _tc_init_edges(
        edge_attr, folded[0]["t"], folded[0]["ae"],
        folded[1]["t"], folded[1]["ae"])
    eps = (ep1, ep2)

    hn = x0
    xin = x0
    for b, fb in enumerate(folded):
        for t in range(3):
            aggr = _sc_edge(src, dst, a_tab, xp, eps[b], zeros_tile)
            last = t == 2
            nxt = folded[b + 1] if (last and b == 0) else fb
            hn, xn, xp, a_tab = _tc_step(
                last, aggr[0], aggr[1], hn, xin,
                fb["ws"], fb["bias"], fb["gru6"], fb["b4"],
                fb["lng"], fb["lnb"], nxt["wn"], nxt["wf"])
            if last:
                xin = xn
                hn = xn

    return _tc_s2s(
        xin, batch2, tuple(s2s_w),
        params["o1_w"], params["o1_b"].reshape(1, 512),
        params["oln_g"].reshape(1, 512), params["oln_b"].reshape(1, 512),
        params["o2_w"], params["o2_b"].reshape(1, 2))


# speedup vs baseline: 7.9359x; 1.4680x over previous
import functools

import jax
import jax.numpy as jnp
from jax import lax
from jax.experimental import pallas as pl
from jax.experimental.pallas import tpu as pltpu
from jax.experimental.pallas import tpu_sc as plsc

N_NODES = 10000
N_EDGES = 160000
EMB = 128
HID = 32
HEADS = 4
ROW = 144
K = 40
N_CHUNKS = N_EDGES // K
N_WORKERS = 32
T_PER_W = (N_CHUNKS + N_WORKERS - 1) // N_WORKERS
N_PAD = 10240
ROWS_PER_TILE = N_PAD // 16

_f32 = jnp.float32
_i32 = jnp.int32


def _sc_edge_body(src_h, dst_h, a_h, xp_h, ep_h, z_h, out_h,
                  srcv, dstv, dsts, arow_d, arow_s, xj, eprow, msg,
                  aggr_sh,
                  sem_src, sem_dst, sem_xj, sem_ad, sem_as, sem_ep, sem_sc):
    cid = lax.axis_index("c")
    sid = lax.axis_index("s")
    wid = sid * 2 + cid

    pltpu.sync_copy(z_h, aggr_sh.at[pl.ds(sid * ROWS_PER_TILE, ROWS_PER_TILE)])

    zero16 = jnp.zeros((16,), _f32)
    for p in range(2):
        for kk in range(3):
            idx = lax.iota(_i32, 16) + (kk * 16)
            mk = idx < K
            for c in range(132, ROW):
                plsc.store_scatter(msg[p], [idx, jnp.full((16,), c, _i32)],
                                   zero16, mask=mk)

    plsc.subcore_barrier()

    def issue_idx(t, p):
        base = (t * N_WORKERS + wid) * K
        pltpu.async_copy(src_h.at[pl.ds(base, K)], srcv[p], sem_src[p])
        pltpu.async_copy(dst_h.at[pl.ds(base, K)], dstv[p], sem_dst[p])

    def wait_idx(p):
        pltpu.make_async_copy(src_h.at[pl.ds(0, K)], srcv[p], sem_src[p]).wait()
        pltpu.make_async_copy(dst_h.at[pl.ds(0, K)], dstv[p], sem_dst[p]).wait()

    def issue_gather(t, p):
        base = (t * N_WORKERS + wid) * K
        pltpu.async_copy(xp_h.at[srcv[p]], xj[p], sem_xj[p])
        pltpu.async_copy(a_h.at[dstv[p]], arow_d[p], sem_ad[p])
        pltpu.async_copy(a_h.at[srcv[p]], arow_s[p], sem_as[p])
        pltpu.async_copy(ep_h.at[pl.ds(base, K)], eprow[p], sem_ep[p])

    def wait_gather(p):
        pltpu.make_async_copy(xp_h.at[srcv[p]], xj[p], sem_xj[p]).wait()
        pltpu.make_async_copy(a_h.at[dstv[p]], arow_d[p], sem_ad[p]).wait()
        pltpu.make_async_copy(a_h.at[srcv[p]], arow_s[p], sem_as[p]).wait()
        pltpu.make_async_copy(ep_h.at[pl.ds(0, K)], eprow[p], sem_ep[p]).wait()

    def wait_scat(p):
        pltpu.make_async_copy(msg[p], aggr_sh.at[dsts[p]], sem_sc[p]).wait()

    def copy_dst_idx(p):
        for kk in range(3):
            idx = lax.iota(_i32, 16) + (kk * 16)
            mk = idx < K
            v = plsc.load_gather(dstv[p], [idx], mask=mk)
            plsc.store_scatter(dsts[p], [idx], v, mask=mk)

    def compute_and_scatter(p):
        for kk in range(3):
            idx = lax.iota(_i32, 16) + (kk * 16)
            mk = idx < K
            for h in range(HEADS):
                ad = plsc.load_gather(arow_d[p], [idx, jnp.full((16,), h, _i32)], mask=mk)
                asr = plsc.load_gather(arow_s[p], [idx, jnp.full((16,), 4 + h, _i32)], mask=mk)
                ae = plsc.load_gather(eprow[p], [idx, jnp.full((16,), 128 + h, _i32)], mask=mk)
                al = ad + asr + ae
                al = jnp.where(al >= 0.0, al, al * 0.2)
                ex = jnp.exp(al)
                plsc.store_scatter(msg[p], [idx, jnp.full((16,), 128 + h, _i32)],
                                   ex, mask=mk)

        def per_edge(k, c2):
            ev = msg[p][k, pl.ds(128, 16)]
            for h in range(HEADS):
                exs = ev[h]
                for c in range(2):
                    sl = pl.ds(h * 32 + c * 16, 16)
                    msg[p][k, sl] = xj[p][k, sl] * eprow[p][k, sl] * exs
            return c2

        lax.fori_loop(0, K, per_edge, 0)
        pltpu.async_copy(msg[p], aggr_sh.at[dsts[p]], sem_sc[p], add=True)

    def half(t, p, first, last_issue):
        wait_gather(p)
        if not first:
            wait_scat(p)
        copy_dst_idx(p)
        if last_issue is None:
            issue_idx(t + 2, p)
        elif last_issue is not False:
            @pl.when(last_issue)
            def _():
                issue_idx(t + 2, p)
        wait_idx(1 - p)
        issue_gather(t + 1, 1 - p)
        compute_and_scatter(p)

    issue_idx(0, 0)
    issue_idx(1, 1)
    wait_idx(0)
    issue_gather(0, 0)

    def pair(i, carry):
        t0 = i * 2

        @pl.when(i == 0)
        def _():
            half(t0, 0, True, None)
            half(t0 + 1, 1, True, i < 61)

        @pl.when(i > 0)
        def _():
            half(t0, 0, False, None)
            half(t0 + 1, 1, False, i < 61)

        return carry

    lax.fori_loop(0, 62, pair, 0)

    wait_gather(0)
    wait_scat(0)
    copy_dst_idx(0)
    compute_and_scatter(0)
    wait_scat(0)
    wait_scat(1)

    plsc.subcore_barrier()

    pltpu.sync_copy(aggr_sh.at[pl.ds(sid * ROWS_PER_TILE, ROWS_PER_TILE)],
                    out_h.at[cid, pl.ds(sid * ROWS_PER_TILE, ROWS_PER_TILE)])


@functools.cache
def _make_sc_edge():
    def _d(shape, dtype):
        return (pltpu.VMEM(shape, dtype), pltpu.VMEM(shape, dtype))

    return functools.partial(
        pl.kernel,
        out_type=jax.ShapeDtypeStruct((2, N_PAD, ROW), _f32),
        mesh=plsc.VectorSubcoreMesh(core_axis_name="c", subcore_axis_name="s"),
        scratch_types=[
            _d((K,), _i32),
            _d((K,), _i32),
            _d((K,), _i32),
            _d((K, 16), _f32),
            _d((K, 16), _f32),
            _d((K, 128), _f32),
            _d((K, ROW), _f32),
            _d((K, ROW), _f32),
            pltpu.VMEM_SHARED((N_PAD, ROW), _f32),
            (pltpu.SemaphoreType.DMA, pltpu.SemaphoreType.DMA),
            (pltpu.SemaphoreType.DMA, pltpu.SemaphoreType.DMA),
            (pltpu.SemaphoreType.DMA, pltpu.SemaphoreType.DMA),
            (pltpu.SemaphoreType.DMA, pltpu.SemaphoreType.DMA),
            (pltpu.SemaphoreType.DMA, pltpu.SemaphoreType.DMA),
            (pltpu.SemaphoreType.DMA, pltpu.SemaphoreType.DMA),
            (pltpu.SemaphoreType.DMA, pltpu.SemaphoreType.DMA),
        ],
        compiler_params=pltpu.CompilerParams(use_tc_tiling_on_sc=False,
                                             needs_layout_passes=False),
    )(_sc_edge_body)


def _sc_edge(*args):
    return _make_sc_edge()(*args)


_RT = 1000


def _celu(x):
    return jnp.where(x > 0.0, x, jnp.exp(x) - 1.0)


def _init_nodes_body(xi_ref, xemb_ref, l0w_ref, l0b_ref, wn_ref, wf_ref,
                     x0_ref, xp_ref, a_ref):
    xi = xi_ref[...]
    ii = lax.broadcasted_iota(_i32, (_RT, 178), 1)
    oh = jnp.zeros((_RT, 178), _f32)
    for j in range(9):
        oh = oh + (xi[:, j:j + 1] == ii).astype(_f32)
    xe = jnp.dot(oh, xemb_ref[...], preferred_element_type=_f32)
    x0 = _celu(jnp.dot(xe, l0w_ref[...], preferred_element_type=_f32)
               + l0b_ref[...])
    x0_ref[...] = x0
    xp_ref[...] = jnp.dot(x0, wn_ref[...], preferred_element_type=_f32)
    a_ref[...] = jnp.dot(x0, wf_ref[...], preferred_element_type=_f32)


def _tc_init_nodes(x_idx, x_emb, l0w, l0b, wn1, wf1):
    n_t = N_NODES // _RT
    w_spec = lambda shp: pl.BlockSpec(shp, lambda i: (0, 0))
    return pl.pallas_call(
        _init_nodes_body,
        grid=(n_t,),
        in_specs=[
            pl.BlockSpec((_RT, 9), lambda i: (i, 0)),
            w_spec((178, EMB)),
            w_spec((EMB, HID)),
            w_spec((1, HID)),
            w_spec((HID, EMB)),
            w_spec((HID, 16)),
        ],
        out_specs=[
            pl.BlockSpec((_RT, HID), lambda i: (i, 0)),
            pl.BlockSpec((_RT, EMB), lambda i: (i, 0)),
            pl.BlockSpec((_RT, 16), lambda i: (i, 0)),
        ],
        out_shape=[
            jax.ShapeDtypeStruct((N_NODES, HID), _f32),
            jax.ShapeDtypeStruct((N_NODES, EMB), _f32),
            jax.ShapeDtypeStruct((N_NODES, 16), _f32),
        ],
    )(x_idx, x_emb, l0w, l0b, wn1, wf1)


_RE = 2000


def _init_edges_body(ea_ref, t1_ref, ae1_ref, t2_ref, ae2_ref,
                     ep1_ref, ep2_ref):
    ea = ea_ref[...]
    ii = lax.broadcasted_iota(_i32, (_RE, 18), 1)
    oh = jnp.zeros((_RE, 18), _f32)
    for j in range(3):
        oh = oh + (ea[:, j:j + 1] == ii).astype(_f32)
    for t_ref, ae_ref, out_ref in ((t1_ref, ae1_ref, ep1_ref),
                                   (t2_ref, ae2_ref, ep2_ref)):
        ep = jnp.dot(oh, t_ref[...], preferred_element_type=_f32)
        ae = jnp.dot(oh, ae_ref[...], preferred_element_type=_f32)
        out_ref[...] = jnp.concatenate([ep, ae], axis=1)


def _tc_init_edges(edge_attr, t1, ae1, t2, ae2):
    n_t = N_EDGES // _RE
    w_spec = lambda shp: pl.BlockSpec(shp, lambda i: (0, 0))
    return pl.pallas_call(
        _init_edges_body,
        grid=(n_t,),
        in_specs=[
            pl.BlockSpec((_RE, 3), lambda i: (i, 0)),
            w_spec((18, EMB)),
            w_spec((18, 16)),
            w_spec((18, EMB)),
            w_spec((18, 16)),
        ],
        out_specs=[
            pl.BlockSpec((_RE, ROW), lambda i: (i, 0)),
            pl.BlockSpec((_RE, ROW), lambda i: (i, 0)),
        ],
        out_shape=[
            jax.ShapeDtypeStruct((N_EDGES, ROW), _f32),
            jax.ShapeDtypeStruct((N_EDGES, ROW), _f32),
        ],
    )(edge_attr, t1, ae1, t2, ae2)


def _step_body(use_xnext, a0_ref, a1_ref, hn_ref, xin_ref,
               ws_ref, bias_ref,
               wir_ref, wiz_ref, win_ref, whr_ref, whz_ref, whn_ref,
               br_ref, bz_ref, bin_ref, bhn_ref,
               lng_ref, lnb_ref, wnn_ref, wfn_ref,
               hn2_ref, xn_ref, xp_ref, a_ref):
    a0 = a0_ref[...]
    a1 = a1_ref[...]
    s = a0[:, 128:ROW] + a1[:, 128:ROW]
    winv = 1.0 / (s + 1e-16)
    r16 = lax.broadcasted_iota(_i32, (16, EMB), 0)
    c16 = lax.broadcasted_iota(_i32, (16, EMB), 1) // 32
    erep = (r16 == c16).astype(_f32)
    wexp = jnp.dot(winv, erep, preferred_element_type=_f32)
    aggr = (a0[:, :128] + a1[:, :128]) * wexp
    mm = _celu(jnp.dot(aggr, ws_ref[...], preferred_element_type=_f32)
               + bias_ref[...])
    hn = hn_ref[...]
    rr = jax.nn.sigmoid(jnp.dot(mm, wir_ref[...], preferred_element_type=_f32)
                        + jnp.dot(hn, whr_ref[...], preferred_element_type=_f32)
                        + br_ref[...])
    zz = jax.nn.sigmoid(jnp.dot(mm, wiz_ref[...], preferred_element_type=_f32)
                        + jnp.dot(hn, whz_ref[...], preferred_element_type=_f32)
                        + bz_ref[...])
    nn = jnp.tanh(jnp.dot(mm, win_ref[...], preferred_element_type=_f32)
                  + bin_ref[...]
                  + rr * (jnp.dot(hn, whn_ref[...], preferred_element_type=_f32)
                          + bhn_ref[...]))
    hn2 = (1.0 - zz) * nn + zz * hn
    mu = jnp.mean(hn2, axis=-1, keepdims=True)
    var = jnp.mean((hn2 - mu) ** 2, axis=-1, keepdims=True)
    cur = (hn2 - mu) / jnp.sqrt(var + 1e-5) * lng_ref[...] + lnb_ref[...]
    xn = xin_ref[...] + cur
    hn2_ref[...] = hn2
    xn_ref[...] = xn
    xsrc = xn if use_xnext else cur
    xp_ref[...] = jnp.dot(xsrc, wnn_ref[...], preferred_element_type=_f32)
    a_ref[...] = jnp.dot(xsrc, wfn_ref[...], preferred_element_type=_f32)


def _tc_step(use_xnext, a0, a1, hn, xin, ws, bias, gru6, b4, lng, lnb,
             wnn, wfn):
    n_t = N_NODES // _RT
    w_spec = lambda shp: pl.BlockSpec(shp, lambda i: (0, 0))
    r_spec = lambda d: pl.BlockSpec((_RT, d), lambda i: (i, 0))
    return pl.pallas_call(
        functools.partial(_step_body, use_xnext),
        grid=(n_t,),
        in_specs=[
            r_spec(ROW), r_spec(ROW), r_spec(HID), r_spec(HID),
            w_spec((EMB, HID)), w_spec((1, HID)),
            w_spec((HID, HID)), w_spec((HID, HID)), w_spec((HID, HID)),
            w_spec((HID, HID)), w_spec((HID, HID)), w_spec((HID, HID)),
            w_spec((1, HID)), w_spec((1, HID)), w_spec((1, HID)),
            w_spec((1, HID)),
            w_spec((1, HID)), w_spec((1, HID)),
            w_spec((HID, EMB)), w_spec((HID, 16)),
        ],
        out_specs=[r_spec(HID), r_spec(HID), r_spec(EMB), r_spec(16)],
        out_shape=[
            jax.ShapeDtypeStruct((N_NODES, HID), _f32),
            jax.ShapeDtypeStruct((N_NODES, HID), _f32),
            jax.ShapeDtypeStruct((N_NODES, EMB), _f32),
            jax.ShapeDtypeStruct((N_NODES, 16), _f32),
        ],
    )(a0, a1, hn, xin, ws, bias, *gru6, *b4, lng, lnb, wnn, wfn)


def _s2s_body(xf_ref, b2_ref,
              wii_ref, wif_ref, wig_ref, wio_ref,
              whi_ref, whf_ref, whg_ref, who_ref,
              bi_ref, bf_ref, bg_ref, bo_ref,
              o1w_ref, o1b_ref, olng_ref, olnb_ref, o2w_ref, o2b_ref,
              out_ref):
    xf = xf_ref[...]
    b2 = b2_ref[...]
    gi = lax.broadcasted_iota(_i32, (N_NODES, 64), 1)
    mask = (b2 == gi).astype(_f32)
    q_star = jnp.zeros((64, 64), _f32)
    h = jnp.zeros((64, HID), _f32)
    c = jnp.zeros((64, HID), _f32)
    for _ in range(3):
        g_i = jax.nn.sigmoid(jnp.dot(q_star, wii_ref[...], preferred_element_type=_f32)
                             + jnp.dot(h, whi_ref[...], preferred_element_type=_f32)
                             + bi_ref[...])
        g_f = jax.nn.sigmoid(jnp.dot(q_star, wif_ref[...], preferred_element_type=_f32)
                             + jnp.dot(h, whf_ref[...], preferred_element_type=_f32)
                             + bf_ref[...])
        g_g = jnp.tanh(jnp.dot(q_star, wig_ref[...], preferred_element_type=_f32)
                       + jnp.dot(h, whg_ref[...], preferred_element_type=_f32)
                       + bg_ref[...])
        g_o = jax.nn.sigmoid(jnp.dot(q_star, wio_ref[...], preferred_element_type=_f32)
                             + jnp.dot(h, who_ref[...], preferred_element_type=_f32)
                             + bo_ref[...])
        c = g_f * c + g_i * g_g
        h = g_o * jnp.tanh(c)
        emat = lax.dot_general(xf, h, (((1,), (1,)), ((), ())),
                               preferred_element_type=_f32)
        e = jnp.sum(mask * emat, axis=1, keepdims=True)
        m = jnp.max(jnp.where(mask > 0.0, emat, -1e30), axis=0,
                    keepdims=True)
        mg = jnp.sum(mask * m, axis=1, keepdims=True)
        ex = jnp.exp(e - mg)
        ssum = jnp.sum(mask * ex, axis=0, keepdims=True)
        sg = jnp.sum(mask * ssum, axis=1, keepdims=True)
        a = ex / (sg + 1e-16)
        r = lax.dot_general(mask, a * xf, (((0,), (0,)), ((), ())),
                            preferred_element_type=_f32)
        q_star = jnp.concatenate([h, r], axis=1)
    hf = jnp.dot(q_star, o1w_ref[...], preferred_element_type=_f32) + o1b_ref[...]
    mu = jnp.mean(hf, axis=-1, keepdims=True)
    var = jnp.mean((hf - mu) ** 2, axis=-1, keepdims=True)
    hf = (hf - mu) / jnp.sqrt(var + 1e-5) * olng_ref[...] + olnb_ref[...]
    hf = jnp.maximum(hf, 0.0)
    out_ref[...] = jnp.dot(hf, o2w_ref[...], preferred_element_type=_f32) + o2b_ref[...]


def _tc_s2s(xf, batch2, s2s_w, o1w, o1b, olng, olnb, o2w, o2b):
    return pl.pallas_call(
        _s2s_body,
        out_shape=jax.ShapeDtypeStruct((64, 2), _f32),
    )(xf, batch2, *s2s_w, o1w, o1b, olng, olnb, o2w, o2b)


def kernel(params, x, edge_index, edge_attr, batch):
    src = edge_index[0]
    dst = edge_index[1]

    blocks = params["blocks"]
    folded = []
    for blk in blocks:
        wn = blk["wn"]
        watt = blk["watt"]
        w_i, w_e, w_j = watt[:, :HID], watt[:, HID:2 * HID], watt[:, 2 * HID:]
        wn3 = wn.reshape(HID, HEADS, HID)
        wf_i = jnp.einsum("khd,hd->kh", wn3, w_i)
        wf_j = jnp.einsum("khd,hd->kh", wn3, w_j)
        wf = jnp.concatenate([wf_i, wf_j, jnp.zeros((HID, 8), _f32)], axis=1)
        t_b = params["e_emb"] @ blk["we"]
        ae_b = jnp.einsum("vhd,hd->vh", t_b.reshape(18, HEADS, HID), w_e)
        ae_b = jnp.concatenate([ae_b, jnp.zeros((18, 12), _f32)], axis=1)
        bih = blk["gru_bih"]
        bhh = blk["gru_bhh"]
        gru6 = tuple(w[:, i * HID:(i + 1) * HID]
                     for w in (blk["gru_wih"], blk["gru_whh"]) for i in range(3))
        gru6 = (gru6[0], gru6[1], gru6[2], gru6[3], gru6[4], gru6[5])
        b4 = (
            (bih[:HID] + bhh[:HID]).reshape(1, HID),
            (bih[HID:2 * HID] + bhh[HID:2 * HID]).reshape(1, HID),
            bih[2 * HID:].reshape(1, HID),
            bhh[2 * HID:].reshape(1, HID),
        )
        folded.append(dict(
            wn=wn, wf=wf, t=t_b, ae=ae_b,
            ws=blk["ws"], bias=blk["bias"].reshape(1, HID),
            gru6=gru6, b4=b4,
            lng=blk["ln_g"].reshape(1, HID), lnb=blk["ln_b"].reshape(1, HID),
        ))

    s2s_w = []
    for w, d in ((params["s2s_wih"], 2 * HID), (params["s2s_whh"], HID)):
        for i in range(4):
            s2s_w.append(w[:, i * HID:(i + 1) * HID])
    for i in range(4):
        s2s_w.append(params["s2s_b"][i * HID:(i + 1) * HID].reshape(1, HID))

    zeros_tile = jnp.zeros((ROWS_PER_TILE, ROW), _f32)
    batch2 = batch.reshape(N_NODES, 1)

    x0, xp, a_tab = _tc_init_nodes(
        x, params["x_emb"], params["lin0_w"],
        params["lin0_b"].reshape(1, HID), folded[0]["wn"], folded[0]["wf"])
    ep1, ep2 = _tc_init_edges(
        edge_attr, folded[0]["t"], folded[0]["ae"],
        folded[1]["t"], folded[1]["ae"])
    eps = (ep1, ep2)

    hn = x0
    xin = x0
    for b, fb in enumerate(folded):
        for t in range(3):
            aggr = _sc_edge(src, dst, a_tab, xp, eps[b], zeros_tile)
            last = t == 2
            nxt = folded[b + 1] if (last and b == 0) else fb
            hn, xn, xp, a_tab = _tc_step(
                last, aggr[0], aggr[1], hn, xin,
                fb["ws"], fb["bias"], fb["gru6"], fb["b4"],
                fb["lng"], fb["lnb"], nxt["wn"], nxt["wf"])
            if last:
                xin = xn
                hn = xn

    return _tc_s2s(
        xin, batch2, tuple(s2s_w),
        params["o1_w"], params["o1_b"].reshape(1, 512),
        params["oln_g"].reshape(1, 512), params["oln_b"].reshape(1, 512),
        params["o2_w"], params["o2_b"].reshape(1, 2))


# speedup vs baseline: 11.4224x; 1.4393x over previous
import functools

import jax
import jax.numpy as jnp
from jax import lax
from jax.experimental import pallas as pl
from jax.experimental.pallas import tpu as pltpu
from jax.experimental.pallas import tpu_sc as plsc

N_NODES = 10000
N_EDGES = 160000
EMB = 128
HID = 32
HEADS = 4
ROW = 144
K = 40
N_CHUNKS = N_EDGES // K
N_WORKERS = 32
T_PER_W = (N_CHUNKS + N_WORKERS - 1) // N_WORKERS
N_PAD = 10240
ROWS_PER_TILE = N_PAD // 16

_f32 = jnp.float32
_i32 = jnp.int32


def _sc_edge_body(src_h, dst_h, a_h, xp_h, ep_h, z_h, out_h,
                  srcv, dstv, dsts, arow_d, arow_s, xj, eprow, msg,
                  aggr_sh,
                  sem_src, sem_dst, sem_xj, sem_ad, sem_as, sem_ep, sem_sc):
    cid = lax.axis_index("c")
    sid = lax.axis_index("s")
    wid = sid * 2 + cid

    pltpu.sync_copy(z_h, aggr_sh.at[pl.ds(sid * ROWS_PER_TILE, ROWS_PER_TILE)])

    zero16 = jnp.zeros((16,), _f32)
    for p in range(2):
        for kk in range(3):
            idx = lax.iota(_i32, 16) + (kk * 16)
            mk = idx < K
            for c in range(132, ROW):
                plsc.store_scatter(msg[p], [idx, jnp.full((16,), c, _i32)],
                                   zero16, mask=mk)

    plsc.subcore_barrier()

    def issue_idx(t, p):
        base = (t * N_WORKERS + wid) * K
        pltpu.async_copy(src_h.at[pl.ds(base, K)], srcv[p], sem_src[p])
        pltpu.async_copy(dst_h.at[pl.ds(base, K)], dstv[p], sem_dst[p])

    def wait_idx(p):
        pltpu.make_async_copy(src_h.at[pl.ds(0, K)], srcv[p], sem_src[p]).wait()
        pltpu.make_async_copy(dst_h.at[pl.ds(0, K)], dstv[p], sem_dst[p]).wait()

    def issue_gather(t, p):
        base = (t * N_WORKERS + wid) * K
        pltpu.async_copy(xp_h.at[srcv[p]], xj[p], sem_xj[p])
        pltpu.async_copy(a_h.at[dstv[p]], arow_d[p], sem_ad[p])
        pltpu.async_copy(a_h.at[srcv[p]], arow_s[p], sem_as[p])
        pltpu.async_copy(ep_h.at[pl.ds(base, K)], eprow[p], sem_ep[p])

    def wait_gather(p):
        pltpu.make_async_copy(xp_h.at[srcv[p]], xj[p], sem_xj[p]).wait()
        pltpu.make_async_copy(a_h.at[dstv[p]], arow_d[p], sem_ad[p]).wait()
        pltpu.make_async_copy(a_h.at[srcv[p]], arow_s[p], sem_as[p]).wait()
        pltpu.make_async_copy(ep_h.at[pl.ds(0, K)], eprow[p], sem_ep[p]).wait()

    def wait_scat(p):
        pltpu.make_async_copy(msg[p], aggr_sh.at[dsts[p]], sem_sc[p]).wait()

    def copy_dst_idx(p):
        for kk in range(3):
            idx = lax.iota(_i32, 16) + (kk * 16)
            mk = idx < K
            v = plsc.load_gather(dstv[p], [idx], mask=mk)
            plsc.store_scatter(dsts[p], [idx], v, mask=mk)

    def compute_and_scatter(p):
        for kk in range(3):
            idx = lax.iota(_i32, 16) + (kk * 16)
            mk = idx < K
            for h in range(HEADS):
                ad = plsc.load_gather(arow_d[p], [idx, jnp.full((16,), h, _i32)], mask=mk)
                asr = plsc.load_gather(arow_s[p], [idx, jnp.full((16,), 4 + h, _i32)], mask=mk)
                ae = plsc.load_gather(eprow[p], [idx, jnp.full((16,), 128 + h, _i32)], mask=mk)
                al = ad + asr + ae
                al = jnp.where(al >= 0.0, al, al * 0.2)
                ex = jnp.exp(al)
                plsc.store_scatter(msg[p], [idx, jnp.full((16,), 128 + h, _i32)],
                                   ex, mask=mk)

        @plsc.parallel_loop(0, K, unroll=4)
        def per_edge(k):
            ev = msg[p][k, pl.ds(128, 16)]
            for h in range(HEADS):
                exs = ev[h]
                for c in range(2):
                    sl = pl.ds(h * 32 + c * 16, 16)
                    msg[p][k, sl] = xj[p][k, sl] * eprow[p][k, sl] * exs
        pltpu.async_copy(msg[p], aggr_sh.at[dsts[p]], sem_sc[p], add=True)

    def half(t, p, first, last_issue):
        wait_gather(p)
        if not first:
            wait_scat(p)
        copy_dst_idx(p)
        if last_issue is None:
            issue_idx(t + 2, p)
        elif last_issue is not False:
            @pl.when(last_issue)
            def _():
                issue_idx(t + 2, p)
        wait_idx(1 - p)
        issue_gather(t + 1, 1 - p)
        compute_and_scatter(p)

    issue_idx(0, 0)
    issue_idx(1, 1)
    wait_idx(0)
    issue_gather(0, 0)

    def pair(i, carry):
        t0 = i * 2

        @pl.when(i == 0)
        def _():
            half(t0, 0, True, None)
            half(t0 + 1, 1, True, i < 61)

        @pl.when(i > 0)
        def _():
            half(t0, 0, False, None)
            half(t0 + 1, 1, False, i < 61)

        return carry

    lax.fori_loop(0, 62, pair, 0)

    wait_gather(0)
    wait_scat(0)
    copy_dst_idx(0)
    compute_and_scatter(0)
    wait_scat(0)
    wait_scat(1)

    plsc.subcore_barrier()

    pltpu.sync_copy(aggr_sh.at[pl.ds(sid * ROWS_PER_TILE, ROWS_PER_TILE)],
                    out_h.at[cid, pl.ds(sid * ROWS_PER_TILE, ROWS_PER_TILE)])


@functools.cache
def _make_sc_edge():
    def _d(shape, dtype):
        return (pltpu.VMEM(shape, dtype), pltpu.VMEM(shape, dtype))

    return functools.partial(
        pl.kernel,
        out_type=jax.ShapeDtypeStruct((2, N_PAD, ROW), _f32),
        mesh=plsc.VectorSubcoreMesh(core_axis_name="c", subcore_axis_name="s"),
        scratch_types=[
            _d((K,), _i32),
            _d((K,), _i32),
            _d((K,), _i32),
            _d((K, 16), _f32),
            _d((K, 16), _f32),
            _d((K, 128), _f32),
            _d((K, ROW), _f32),
            _d((K, ROW), _f32),
            pltpu.VMEM_SHARED((N_PAD, ROW), _f32),
            (pltpu.SemaphoreType.DMA, pltpu.SemaphoreType.DMA),
            (pltpu.SemaphoreType.DMA, pltpu.SemaphoreType.DMA),
            (pltpu.SemaphoreType.DMA, pltpu.SemaphoreType.DMA),
            (pltpu.SemaphoreType.DMA, pltpu.SemaphoreType.DMA),
            (pltpu.SemaphoreType.DMA, pltpu.SemaphoreType.DMA),
            (pltpu.SemaphoreType.DMA, pltpu.SemaphoreType.DMA),
            (pltpu.SemaphoreType.DMA, pltpu.SemaphoreType.DMA),
        ],
        compiler_params=pltpu.CompilerParams(use_tc_tiling_on_sc=False,
                                             needs_layout_passes=False),
    )(_sc_edge_body)


def _sc_edge(*args):
    return _make_sc_edge()(*args)


_RT = 1000


def _celu(x):
    return jnp.where(x > 0.0, x, jnp.exp(x) - 1.0)


def _init_nodes_body(xi_ref, xemb_ref, l0w_ref, l0b_ref, wn_ref, wf_ref,
                     x0_ref, xp_ref, a_ref):
    xi = xi_ref[...]
    ii = lax.broadcasted_iota(_i32, (_RT, 178), 1)
    oh = jnp.zeros((_RT, 178), _f32)
    for j in range(9):
        oh = oh + (xi[:, j:j + 1] == ii).astype(_f32)
    xe = jnp.dot(oh, xemb_ref[...], preferred_element_type=_f32)
    x0 = _celu(jnp.dot(xe, l0w_ref[...], preferred_element_type=_f32)
               + l0b_ref[...])
    x0_ref[...] = x0
    xp_ref[...] = jnp.dot(x0, wn_ref[...], preferred_element_type=_f32)
    a_ref[...] = jnp.dot(x0, wf_ref[...], preferred_element_type=_f32)


def _tc_init_nodes(x_idx, x_emb, l0w, l0b, wn1, wf1):
    n_t = N_NODES // _RT
    w_spec = lambda shp: pl.BlockSpec(shp, lambda i: (0, 0))
    return pl.pallas_call(
        _init_nodes_body,
        grid=(n_t,),
        in_specs=[
            pl.BlockSpec((_RT, 9), lambda i: (i, 0)),
            w_spec((178, EMB)),
            w_spec((EMB, HID)),
            w_spec((1, HID)),
            w_spec((HID, EMB)),
            w_spec((HID, 16)),
        ],
        out_specs=[
            pl.BlockSpec((_RT, HID), lambda i: (i, 0)),
            pl.BlockSpec((_RT, EMB), lambda i: (i, 0)),
            pl.BlockSpec((_RT, 16), lambda i: (i, 0)),
        ],
        out_shape=[
            jax.ShapeDtypeStruct((N_NODES, HID), _f32),
            jax.ShapeDtypeStruct((N_NODES, EMB), _f32),
            jax.ShapeDtypeStruct((N_NODES, 16), _f32),
        ],
    )(x_idx, x_emb, l0w, l0b, wn1, wf1)


_RE = 2000


def _init_edges_body(ea_ref, t1_ref, ae1_ref, t2_ref, ae2_ref,
                     ep1_ref, ep2_ref):
    ea = ea_ref[...]
    ii = lax.broadcasted_iota(_i32, (_RE, 18), 1)
    oh = jnp.zeros((_RE, 18), _f32)
    for j in range(3):
        oh = oh + (ea[:, j:j + 1] == ii).astype(_f32)
    for t_ref, ae_ref, out_ref in ((t1_ref, ae1_ref, ep1_ref),
                                   (t2_ref, ae2_ref, ep2_ref)):
        ep = jnp.dot(oh, t_ref[...], preferred_element_type=_f32)
        ae = jnp.dot(oh, ae_ref[...], preferred_element_type=_f32)
        out_ref[...] = jnp.concatenate([ep, ae], axis=1)


def _tc_init_edges(edge_attr, t1, ae1, t2, ae2):
    n_t = N_EDGES // _RE
    w_spec = lambda shp: pl.BlockSpec(shp, lambda i: (0, 0))
    return pl.pallas_call(
        _init_edges_body,
        grid=(n_t,),
        in_specs=[
            pl.BlockSpec((_RE, 3), lambda i: (i, 0)),
            w_spec((18, EMB)),
            w_spec((18, 16)),
            w_spec((18, EMB)),
            w_spec((18, 16)),
        ],
        out_specs=[
            pl.BlockSpec((_RE, ROW), lambda i: (i, 0)),
            pl.BlockSpec((_RE, ROW), lambda i: (i, 0)),
        ],
        out_shape=[
            jax.ShapeDtypeStruct((N_EDGES, ROW), _f32),
            jax.ShapeDtypeStruct((N_EDGES, ROW), _f32),
        ],
    )(edge_attr, t1, ae1, t2, ae2)


def _step_body(use_xnext, a0_ref, a1_ref, hn_ref, xin_ref,
               ws_ref, bias_ref,
               wir_ref, wiz_ref, win_ref, whr_ref, whz_ref, whn_ref,
               br_ref, bz_ref, bin_ref, bhn_ref,
               lng_ref, lnb_ref, wnn_ref, wfn_ref,
               hn2_ref, xn_ref, xp_ref, a_ref):
    a0 = a0_ref[0]
    a1 = a1_ref[0]
    s = a0[:, 128:ROW] + a1[:, 128:ROW]
    winv = 1.0 / (s + 1e-16)
    r16 = lax.broadcasted_iota(_i32, (16, EMB), 0)
    c16 = lax.broadcasted_iota(_i32, (16, EMB), 1) // 32
    erep = (r16 == c16).astype(_f32)
    wexp = jnp.dot(winv, erep, preferred_element_type=_f32)
    aggr = (a0[:, :128] + a1[:, :128]) * wexp
    mm = _celu(jnp.dot(aggr, ws_ref[...], preferred_element_type=_f32)
               + bias_ref[...])
    hn = hn_ref[...]
    rr = jax.nn.sigmoid(jnp.dot(mm, wir_ref[...], preferred_element_type=_f32)
                        + jnp.dot(hn, whr_ref[...], preferred_element_type=_f32)
                        + br_ref[...])
    zz = jax.nn.sigmoid(jnp.dot(mm, wiz_ref[...], preferred_element_type=_f32)
                        + jnp.dot(hn, whz_ref[...], preferred_element_type=_f32)
                        + bz_ref[...])
    nn = jnp.tanh(jnp.dot(mm, win_ref[...], preferred_element_type=_f32)
                  + bin_ref[...]
                  + rr * (jnp.dot(hn, whn_ref[...], preferred_element_type=_f32)
                          + bhn_ref[...]))
    hn2 = (1.0 - zz) * nn + zz * hn
    mu = jnp.mean(hn2, axis=-1, keepdims=True)
    var = jnp.mean((hn2 - mu) ** 2, axis=-1, keepdims=True)
    cur = (hn2 - mu) / jnp.sqrt(var + 1e-5) * lng_ref[...] + lnb_ref[...]
    xn = xin_ref[...] + cur
    hn2_ref[...] = hn2
    xn_ref[...] = xn
    xsrc = xn if use_xnext else cur
    xp_ref[...] = jnp.dot(xsrc, wnn_ref[...], preferred_element_type=_f32)
    a_ref[...] = jnp.dot(xsrc, wfn_ref[...], preferred_element_type=_f32)


def _tc_step(use_xnext, aggr, hn, xin, ws, bias, gru6, b4, lng, lnb,
             wnn, wfn):
    n_t = N_NODES // _RT
    w_spec = lambda shp: pl.BlockSpec(shp, lambda i: (0, 0))
    r_spec = lambda d: pl.BlockSpec((_RT, d), lambda i: (i, 0))
    return pl.pallas_call(
        functools.partial(_step_body, use_xnext),
        grid=(n_t,),
        in_specs=[
            pl.BlockSpec((1, _RT, ROW), lambda i: (0, i, 0)),
            pl.BlockSpec((1, _RT, ROW), lambda i: (1, i, 0)),
            r_spec(HID), r_spec(HID),
            w_spec((EMB, HID)), w_spec((1, HID)),
            w_spec((HID, HID)), w_spec((HID, HID)), w_spec((HID, HID)),
            w_spec((HID, HID)), w_spec((HID, HID)), w_spec((HID, HID)),
            w_spec((1, HID)), w_spec((1, HID)), w_spec((1, HID)),
            w_spec((1, HID)),
            w_spec((1, HID)), w_spec((1, HID)),
            w_spec((HID, EMB)), w_spec((HID, 16)),
        ],
        out_specs=[r_spec(HID), r_spec(HID), r_spec(EMB), r_spec(16)],
        out_shape=[
            jax.ShapeDtypeStruct((N_NODES, HID), _f32),
            jax.ShapeDtypeStruct((N_NODES, HID), _f32),
            jax.ShapeDtypeStruct((N_NODES, EMB), _f32),
            jax.ShapeDtypeStruct((N_NODES, 16), _f32),
        ],
    )(aggr, aggr, hn, xin, ws, bias, *gru6, *b4, lng, lnb, wnn, wfn)


def _s2s_body(xf_ref, b2_ref,
              wii_ref, wif_ref, wig_ref, wio_ref,
              whi_ref, whf_ref, whg_ref, who_ref,
              bi_ref, bf_ref, bg_ref, bo_ref,
              o1w_ref, o1b_ref, olng_ref, olnb_ref, o2w_ref, o2b_ref,
              out_ref):
    xf = xf_ref[...]
    b2 = b2_ref[...]
    gi = lax.broadcasted_iota(_i32, (N_NODES, 64), 1)
    mask = (b2 == gi).astype(_f32)
    q_star = jnp.zeros((64, 64), _f32)
    h = jnp.zeros((64, HID), _f32)
    c = jnp.zeros((64, HID), _f32)
    for _ in range(3):
        g_i = jax.nn.sigmoid(jnp.dot(q_star, wii_ref[...], preferred_element_type=_f32)
                             + jnp.dot(h, whi_ref[...], preferred_element_type=_f32)
                             + bi_ref[...])
        g_f = jax.nn.sigmoid(jnp.dot(q_star, wif_ref[...], preferred_element_type=_f32)
                             + jnp.dot(h, whf_ref[...], preferred_element_type=_f32)
                             + bf_ref[...])
        g_g = jnp.tanh(jnp.dot(q_star, wig_ref[...], preferred_element_type=_f32)
                       + jnp.dot(h, whg_ref[...], preferred_element_type=_f32)
                       + bg_ref[...])
        g_o = jax.nn.sigmoid(jnp.dot(q_star, wio_ref[...], preferred_element_type=_f32)
                             + jnp.dot(h, who_ref[...], preferred_element_type=_f32)
                             + bo_ref[...])
        c = g_f * c + g_i * g_g
        h = g_o * jnp.tanh(c)
        emat = lax.dot_general(xf, h, (((1,), (1,)), ((), ())),
                               preferred_element_type=_f32)
        e = jnp.sum(mask * emat, axis=1, keepdims=True)
        m = jnp.max(jnp.where(mask > 0.0, emat, -1e30), axis=0,
                    keepdims=True)
        mg = jnp.sum(mask * m, axis=1, keepdims=True)
        ex = jnp.exp(e - mg)
        ssum = jnp.sum(mask * ex, axis=0, keepdims=True)
        sg = jnp.sum(mask * ssum, axis=1, keepdims=True)
        a = ex / (sg + 1e-16)
        r = lax.dot_general(mask, a * xf, (((0,), (0,)), ((), ())),
                            preferred_element_type=_f32)
        q_star = jnp.concatenate([h, r], axis=1)
    hf = jnp.dot(q_star, o1w_ref[...], preferred_element_type=_f32) + o1b_ref[...]
    mu = jnp.mean(hf, axis=-1, keepdims=True)
    var = jnp.mean((hf - mu) ** 2, axis=-1, keepdims=True)
    hf = (hf - mu) / jnp.sqrt(var + 1e-5) * olng_ref[...] + olnb_ref[...]
    hf = jnp.maximum(hf, 0.0)
    out_ref[...] = jnp.dot(hf, o2w_ref[...], preferred_element_type=_f32) + o2b_ref[...]


def _tc_s2s(xf, batch2, s2s_w, o1w, o1b, olng, olnb, o2w, o2b):
    return pl.pallas_call(
        _s2s_body,
        out_shape=jax.ShapeDtypeStruct((64, 2), _f32),
    )(xf, batch2, *s2s_w, o1w, o1b, olng, olnb, o2w, o2b)


def kernel(params, x, edge_index, edge_attr, batch):
    src = edge_index[0]
    dst = edge_index[1]

    blocks = params["blocks"]
    folded = []
    for blk in blocks:
        wn = blk["wn"]
        watt = blk["watt"]
        w_i, w_e, w_j = watt[:, :HID], watt[:, HID:2 * HID], watt[:, 2 * HID:]
        wn3 = wn.reshape(HID, HEADS, HID)
        wf_i = jnp.einsum("khd,hd->kh", wn3, w_i)
        wf_j = jnp.einsum("khd,hd->kh", wn3, w_j)
        wf = jnp.concatenate([wf_i, wf_j, jnp.zeros((HID, 8), _f32)], axis=1)
        t_b = params["e_emb"] @ blk["we"]
        ae_b = jnp.einsum("vhd,hd->vh", t_b.reshape(18, HEADS, HID), w_e)
        ae_b = jnp.concatenate([ae_b, jnp.zeros((18, 12), _f32)], axis=1)
        bih = blk["gru_bih"]
        bhh = blk["gru_bhh"]
        gru6 = tuple(w[:, i * HID:(i + 1) * HID]
                     for w in (blk["gru_wih"], blk["gru_whh"]) for i in range(3))
        gru6 = (gru6[0], gru6[1], gru6[2], gru6[3], gru6[4], gru6[5])
        b4 = (
            (bih[:HID] + bhh[:HID]).reshape(1, HID),
            (bih[HID:2 * HID] + bhh[HID:2 * HID]).reshape(1, HID),
            bih[2 * HID:].reshape(1, HID),
            bhh[2 * HID:].reshape(1, HID),
        )
        folded.append(dict(
            wn=wn, wf=wf, t=t_b, ae=ae_b,
            ws=blk["ws"], bias=blk["bias"].reshape(1, HID),
            gru6=gru6, b4=b4,
            lng=blk["ln_g"].reshape(1, HID), lnb=blk["ln_b"].reshape(1, HID),
        ))

    s2s_w = []
    for w, d in ((params["s2s_wih"], 2 * HID), (params["s2s_whh"], HID)):
        for i in range(4):
            s2s_w.append(w[:, i * HID:(i + 1) * HID])
    for i in range(4):
        s2s_w.append(params["s2s_b"][i * HID:(i + 1) * HID].reshape(1, HID))

    zeros_tile = jnp.zeros((ROWS_PER_TILE, ROW), _f32)
    batch2 = batch.reshape(N_NODES, 1)

    x0, xp, a_tab = _tc_init_nodes(
        x, params["x_emb"], params["lin0_w"],
        params["lin0_b"].reshape(1, HID), folded[0]["wn"], folded[0]["wf"])
    ep1, ep2 = _tc_init_edges(
        edge_attr, folded[0]["t"], folded[0]["ae"],
        folded[1]["t"], folded[1]["ae"])
    eps = (ep1, ep2)

    hn = x0
    xin = x0
    for b, fb in enumerate(folded):
        for t in range(3):
            aggr = _sc_edge(src, dst, a_tab, xp, eps[b], zeros_tile)
            last = t == 2
            nxt = folded[b + 1] if (last and b == 0) else fb
            hn, xn, xp, a_tab = _tc_step(
                last, aggr, hn, xin,
                fb["ws"], fb["bias"], fb["gru6"], fb["b4"],
                fb["lng"], fb["lnb"], nxt["wn"], nxt["wf"])
            if last:
                xin = xn
                hn = xn

    return _tc_s2s(
        xin, batch2, tuple(s2s_w),
        params["o1_w"], params["o1_b"].reshape(1, 512),
        params["oln_g"].reshape(1, 512), params["oln_b"].reshape(1, 512),
        params["o2_w"], params["o2_b"].reshape(1, 2))
